# baseline (speedup 1.0000x reference)
import functools

import jax
import jax.numpy as jnp
from jax import lax
from jax.experimental import pallas as pl
from jax.experimental.pallas import tpu as pltpu
from jax.experimental.pallas import tpu_sc as plsc

_L = 16
_NSUB = 32
_BLK_M = 1024


def _proj_body(x_ref, w_ref, v_ref, f_ref, a_ref):
    xb = x_ref[...]
    f_ref[...] = jnp.dot(xb, w_ref[...], preferred_element_type=jnp.float32,
                         precision=lax.Precision.HIGHEST)
    a_ref[...] = jnp.dot(xb, v_ref[...], preferred_element_type=jnp.float32,
                         precision=lax.Precision.HIGHEST)


def _proj(x_p, w_cat, v_mat):
    m, k = x_p.shape
    n = w_cat.shape[1]
    na = v_mat.shape[1]
    grid = (m // _BLK_M,)
    return pl.pallas_call(
        _proj_body,
        grid=grid,
        in_specs=[
            pl.BlockSpec((_BLK_M, k), lambda i: (i, 0)),
            pl.BlockSpec((k, n), lambda i: (0, 0)),
            pl.BlockSpec((k, na), lambda i: (0, 0)),
        ],
        out_specs=[
            pl.BlockSpec((_BLK_M, n), lambda i: (i, 0)),
            pl.BlockSpec((_BLK_M, na), lambda i: (i, 0)),
        ],
        out_shape=[
            jax.ShapeDtypeStruct((m, n), jnp.float32),
            jax.ShapeDtypeStruct((m, na), jnp.float32),
        ],
    )(x_p, w_cat, v_mat)


def _attvec_body(wg_ref, am_ref, o_ref):
    o_ref[...] = jnp.dot(wg_ref[...], am_ref[...],
                         preferred_element_type=jnp.float32,
                         precision=lax.Precision.HIGHEST)


def _attvec(w_gat, att_mat):
    k, n = w_gat.shape
    na = att_mat.shape[1]
    return pl.pallas_call(
        _attvec_body,
        in_specs=[
            pl.BlockSpec((k, n), lambda: (0, 0)),
            pl.BlockSpec((n, na), lambda: (0, 0)),
        ],
        out_specs=pl.BlockSpec((k, na), lambda: (0, 0)),
        out_shape=jax.ShapeDtypeStruct((k, na), jnp.float32),
    )(w_gat, att_mat)


def _fuse_body(oe_ref, ft_ref, d2_ref, ws_ref, bg_ref, bt_ref, wf_ref,
               bf_ref, o_ref):
    agg = oe_ref[...]
    ft = ft_ref[...]
    d2 = d2_ref[...]
    ws = ws_ref[...]
    C = 256
    gcn = agg[:, :C] + d2 * ft[:, :C] + bg_ref[...]
    parts = [gcn]
    for h in range(4):
        lo = C + h * C
        gat_h = (agg[:, lo:lo + C] + ws[:, h:h + 1] * ft[:, lo:lo + C]
                 + bt_ref[:, h * C:(h + 1) * C])
        parts.append(gat_h)
    xcat = jnp.concatenate(parts, axis=1)
    acc = jnp.dot(xcat, wf_ref[...], preferred_element_type=jnp.float32,
                         precision=lax.Precision.HIGHEST)
    acc = acc + bf_ref[...]
    o_ref[...] = jnp.where(acc > 0, acc, jnp.exp(acc) - 1.0)


def _fuse(out_edge, feat, dinv2, w_self, b_gcn, b_gat, w_fuse, b_fuse):
    m, k = out_edge.shape
    n = w_fuse.shape[1]
    grid = (m // _BLK_M,)
    return pl.pallas_call(
        _fuse_body,
        grid=grid,
        in_specs=[
            pl.BlockSpec((_BLK_M, k), lambda i: (i, 0)),
            pl.BlockSpec((_BLK_M, k), lambda i: (i, 0)),
            pl.BlockSpec((_BLK_M, 1), lambda i: (i, 0)),
            pl.BlockSpec((_BLK_M, 4), lambda i: (i, 0)),
            pl.BlockSpec((1, 256), lambda i: (0, 0)),
            pl.BlockSpec((1, 1024), lambda i: (0, 0)),
            pl.BlockSpec((k, n), lambda i: (0, 0)),
            pl.BlockSpec((1, n), lambda i: (0, 0)),
        ],
        out_specs=pl.BlockSpec((_BLK_M, n), lambda i: (i, 0)),
        out_shape=jax.ShapeDtypeStruct((m, n), jnp.float32),
    )(out_edge, feat, dinv2, w_self, b_gcn, b_gat, w_fuse, b_fuse)


_SC_PARAMS = pltpu.CompilerParams(use_tc_tiling_on_sc=False,
                                  needs_layout_passes=False)


def _sc_deg_asum(np_, e_pad, asrc_f, adst_f, row_p, col_p, zrows):
    CH = 128
    slice_len = e_pad // _NSUB
    nchunks = slice_len // CH
    stripe = np_ // 16
    mesh = plsc.VectorSubcoreMesh(core_axis_name="c", subcore_axis_name="s")

    @functools.partial(
        pl.kernel, mesh=mesh,
        compiler_params=_SC_PARAMS,
        out_type=jax.ShapeDtypeStruct((2 * np_, 8), jnp.float32),
        scratch_types=[
            pltpu.VMEM((4 * np_,), jnp.float32),
            pltpu.VMEM((4 * np_,), jnp.float32),
            pltpu.VMEM((CH,), jnp.int32),
            pltpu.VMEM((CH,), jnp.int32),
            pltpu.VMEM((CH, 8), jnp.float32),
            pltpu.VMEM_SHARED((np_, 8), jnp.float32),
        ],
    )
    def k(asrc_h, adst_h, row_h, col_h, z_h, out_h,
          asrc_v, adst_v, rst, cst, sbuf, acc):
        c = lax.axis_index("c")
        s = lax.axis_index("s")
        pltpu.sync_copy(z_h, acc.at[pl.ds(s * stripe, stripe)])
        pltpu.sync_copy(asrc_h, asrc_v)
        pltpu.sync_copy(adst_h, adst_v)
        plsc.subcore_barrier()

        wid = s * 2 + c
        base0 = wid * slice_len
        lane = lax.iota(jnp.int32, 16)
        ones = jnp.ones((16,), jnp.float32)

        def chunk(i, carry):
            b = base0 + i * CH
            pltpu.sync_copy(row_h.at[pl.ds(b, CH)], rst)
            pltpu.sync_copy(col_h.at[pl.ds(b, CH)], cst)
            for g in range(CH // 16):
                rv = rst[pl.ds(g * 16, 16)]
                cv = cst[pl.ds(g * 16, 16)]
                plsc.store_scatter(sbuf, [g * 16 + lane, 0 * lane], ones)
                for h in range(4):
                    sv = plsc.load_gather(asrc_v, [rv + h * np_])
                    dv = plsc.load_gather(adst_v, [cv + h * np_])
                    z = sv + dv
                    al = jnp.maximum(z, 0.0) + 0.2 * jnp.minimum(z, 0.0)
                    p = jnp.exp(al - jnp.maximum(dv, 0.0))
                    plsc.store_scatter(
                        sbuf, [g * 16 + lane, 0 * lane + 1 + h], p)
            pltpu.sync_copy(sbuf, acc.at[cst], add=True)
            return carry

        lax.fori_loop(0, nchunks, chunk, 0)
        plsc.subcore_barrier()
        pltpu.sync_copy(acc.at[pl.ds(s * stripe, stripe)],
                        out_h.at[pl.ds(c * np_ + s * stripe, stripe)])

    return k(asrc_f, adst_f, row_p, col_p, zrows)


def _sc_edge_weights(np_, e_pad, asrc_f, adst_f, rinv_f, dinv_p,
                     row_p, col_p):
    CH = 1024
    slice_len = e_pad // _NSUB
    nchunks = slice_len // CH
    mesh = plsc.VectorSubcoreMesh(core_axis_name="c", subcore_axis_name="s")

    @functools.partial(
        pl.kernel, mesh=mesh,
        compiler_params=_SC_PARAMS,
        out_type=jax.ShapeDtypeStruct((5 * e_pad,), jnp.float32),
        scratch_types=[
            pltpu.VMEM((np_,), jnp.float32),
            pltpu.VMEM((np_,), jnp.float32),
            pltpu.VMEM((np_,), jnp.float32),
            pltpu.VMEM((CH,), jnp.int32),
            pltpu.VMEM((CH,), jnp.int32),
            pltpu.VMEM((CH,), jnp.float32),
        ],
    )
    def k(asrc_h, adst_h, rinv_h, dinv_h, row_h, col_h, out_h,
          ta, tb, tc, rst, cst, obuf):
        c = lax.axis_index("c")
        s = lax.axis_index("s")
        wid = s * 2 + c
        base0 = wid * slice_len
        lane = lax.iota(jnp.int32, 16)

        for ph in range(5):
            if ph == 0:
                pltpu.sync_copy(dinv_h, ta)
            else:
                h = ph - 1
                pltpu.sync_copy(asrc_h.at[pl.ds(h * np_, np_)], ta)
                pltpu.sync_copy(adst_h.at[pl.ds(h * np_, np_)], tb)
                pltpu.sync_copy(rinv_h.at[pl.ds(h * np_, np_)], tc)

            def chunk(i, carry, ph=ph):
                b = base0 + i * CH
                pltpu.sync_copy(row_h.at[pl.ds(b, CH)], rst)
                pltpu.sync_copy(col_h.at[pl.ds(b, CH)], cst)

                def grp(g, carry1):
                    rv = rst[pl.ds(g * 16, 16)]
                    cv = cst[pl.ds(g * 16, 16)]
                    if ph == 0:
                        wv = (plsc.load_gather(ta, [rv])
                              * plsc.load_gather(ta, [cv]))
                    else:
                        sv = plsc.load_gather(ta, [rv])
                        dv = plsc.load_gather(tb, [cv])
                        ri = plsc.load_gather(tc, [cv])
                        z = sv + dv
                        al = jnp.maximum(z, 0.0) + 0.2 * jnp.minimum(z, 0.0)
                        wv = jnp.exp(al - jnp.maximum(dv, 0.0)) * ri
                    plsc.store_scatter(obuf, [g * 16 + lane], wv)
                    return carry1

                lax.fori_loop(0, CH // 16, grp, 0)
                pltpu.sync_copy(obuf, out_h.at[pl.ds(ph * e_pad + b, CH)])
                return carry

            lax.fori_loop(0, nchunks, chunk, 0)

    return k(asrc_f, adst_f, rinv_f, dinv_p, row_p, col_p)


def _sc_aggregate(np_, e_pad, wgt8, feat, row_p, col_p, zrows):
    W = 384
    D = 1280
    NPASS = np_ // (W * 2)
    CH = 2048
    slice_len = e_pad // 16
    nchunks = slice_len // CH
    cap = 2112
    stripe = W // 16
    mesh = plsc.VectorSubcoreMesh(core_axis_name="c", subcore_axis_name="s")

    @functools.partial(
        pl.kernel, mesh=mesh,
        compiler_params=_SC_PARAMS,
        out_type=jax.ShapeDtypeStruct((np_, D), jnp.float32),
        scratch_types=[
            pltpu.VMEM((CH,), jnp.int32),
            pltpu.VMEM((CH,), jnp.int32),
            pltpu.VMEM((cap,), jnp.int32),
            pltpu.VMEM((cap,), jnp.int32),
            pltpu.VMEM((cap,), jnp.int32),
            pltpu.VMEM((16, 8), jnp.float32),
            pltpu.VMEM((16, 8), jnp.float32),
            pltpu.VMEM((16, D), jnp.float32),
            pltpu.VMEM((16, D), jnp.float32),
            pltpu.VMEM((16, D), jnp.float32),
            pltpu.VMEM((16, D), jnp.float32),
            pltpu.VMEM_SHARED((W, D), jnp.float32),
            pltpu.SemaphoreType.DMA,
            pltpu.SemaphoreType.DMA,
            pltpu.SemaphoreType.DMA,
            pltpu.SemaphoreType.DMA,
            pltpu.SemaphoreType.DMA,
            pltpu.SemaphoreType.DMA,
        ],
    )
    def k(wgt_h, feat_h, row_h, col_h, z_h, out_h,
          rst, cst, rbuf, cbuf, ebuf, wbufA, wbufB, fbufA, fbufB,
          sbufA, sbufB, acc, fsemA, fsemB, wsemA, wsemB, ssemA, ssemB):
        c = lax.axis_index("c")
        s = lax.axis_index("s")
        lane = lax.iota(jnp.int32, 16)
        base0 = s * slice_len

        def zinit(i, carry):
            rbuf[pl.ds(i * 16, 16)] = 0 * lane
            cbuf[pl.ds(i * 16, 16)] = 0 * lane
            ebuf[pl.ds(i * 16, 16)] = 0 * lane
            return carry

        lax.fori_loop(0, cap // 16, zinit, 0)

        def scale(wb, fb, sb, j, cnt):
            valid = ((j * 16 + lane) < cnt).astype(jnp.float32)
            for kk in range(5):
                wk = plsc.load_gather(wb, [lane, 0 * lane + kk])
                plsc.store_scatter(wb, [lane, 0 * lane + kk], wk * valid)

            def edge(jj, carry3):
                jv = 0 * lane + jj
                gs = plsc.load_gather(wb, [jv, 0 * lane])
                hs = [plsc.load_gather(wb, [jv, 0 * lane + 1 + h])
                      for h in range(4)]
                for v in range(D // 16):
                    kv = lane + v * 16
                    wv = gs if v < 16 else hs[(v - 16) // 16]
                    xv = plsc.load_gather(fb, [jv, kv])
                    plsc.store_scatter(sb, [jv, kv], xv * wv)
                return carry3

            lax.fori_loop(0, 16, edge, 0)

        def issue(jg, wb, fb, wsem, fsem):
            ev = ebuf[pl.ds(jg * 16, 16)]
            rv = rbuf[pl.ds(jg * 16, 16)]
            pltpu.async_copy(wgt_h.at[ev], wb, wsem)
            pltpu.async_copy(feat_h.at[rv], fb, fsem)

        def wait_g(wb, fb, wsem, fsem):
            ev0 = ebuf[pl.ds(0, 16)]
            rv0 = rbuf[pl.ds(0, 16)]
            pltpu.make_async_copy(wgt_h.at[ev0], wb, wsem).wait()
            pltpu.make_async_copy(feat_h.at[rv0], fb, fsem).wait()

        def wait_s(sb, ssem):
            relc0 = cbuf[pl.ds(0, 16)]
            pltpu.make_async_copy(sb, acc.at[relc0], ssem).wait()

        def one_pass(p, carry):
            wlo = (p * 2 + c) * W
            pltpu.sync_copy(z_h, acc.at[pl.ds(s * stripe, stripe)])
            plsc.subcore_barrier()

            def chunk(i, carry1):
                b = base0 + i * CH
                pltpu.sync_copy(row_h.at[pl.ds(b, CH)], rst)
                pltpu.sync_copy(col_h.at[pl.ds(b, CH)], cst)

                def scan_grp(g, cnt):
                    rv = rst[pl.ds(g * 16, 16)]
                    cv = cst[pl.ds(g * 16, 16)]
                    rel = cv - wlo
                    m = (rel >= 0) & (rel < W)
                    plsc.store_compressed(rbuf.at[pl.ds(cnt, 16)], rv,
                                          mask=m)
                    plsc.store_compressed(cbuf.at[pl.ds(cnt, 16)], rel,
                                          mask=m)
                    plsc.store_compressed(ebuf.at[pl.ds(cnt, 16)],
                                          b + g * 16 + lane, mask=m)
                    return cnt + jnp.max(
                        plsc.all_reduce_population_count(m))

                cnt = lax.fori_loop(0, CH // 16, scan_grp, jnp.int32(0))
                rbuf[pl.ds(cnt, 16)] = 0 * lane
                cbuf[pl.ds(cnt, 16)] = 0 * lane
                ebuf[pl.ds(cnt, 16)] = 0 * lane

                issue(0, wbufA, fbufA, wsemA, fsemA)
                trips = (cnt + 31) // 32

                def grp2(jp, carry2):
                    jA = jp * 2
                    jB = jp * 2 + 1
                    issue(jB, wbufB, fbufB, wsemB, fsemB)
                    wait_g(wbufA, fbufA, wsemA, fsemA)

                    @pl.when(jp > 0)
                    def _():
                        wait_s(sbufA, ssemA)

                    scale(wbufA, fbufA, sbufA, jA, cnt)
                    relcA = cbuf[pl.ds(jA * 16, 16)]
                    pltpu.async_copy(sbufA, acc.at[relcA], ssemA,
                                     add=True)
                    issue(jA + 2, wbufA, fbufA, wsemA, fsemA)
                    wait_g(wbufB, fbufB, wsemB, fsemB)

                    @pl.when(jp > 0)
                    def _():
                        wait_s(sbufB, ssemB)

                    scale(wbufB, fbufB, sbufB, jB, cnt)
                    relcB = cbuf[pl.ds(jB * 16, 16)]
                    pltpu.async_copy(sbufB, acc.at[relcB], ssemB,
                                     add=True)
                    return carry2

                lax.fori_loop(0, trips, grp2, 0)

                @pl.when(trips > 0)
                def _():
                    wait_s(sbufA, ssemA)
                    wait_s(sbufB, ssemB)

                wait_g(wbufA, fbufA, wsemA, fsemA)
                return carry1

            lax.fori_loop(0, nchunks, chunk, 0)
            plsc.subcore_barrier()
            pltpu.sync_copy(acc.at[pl.ds(s * stripe, stripe)],
                            out_h.at[pl.ds(wlo + s * stripe, stripe)])
            return carry

        lax.fori_loop(0, NPASS, one_pass, 0)

    return k(wgt8, feat, row_p, col_p, zrows)


def kernel(x, edge_index, W_gcn, b_gcn, W_gat, att_src, att_dst, b_gat,
           W_fuse, b_fuse):
    n, d_in = x.shape
    H = att_src.shape[1]
    C = att_src.shape[2]
    E = edge_index.shape[1]
    NP = 12288
    EP = ((E + 4095) // 4096) * 4096

    x_p = jnp.pad(x, ((0, NP - n), (0, 0)))
    row_p = jnp.pad(edge_index[0], (0, EP - E)).astype(jnp.int32)
    col_p = jnp.pad(edge_index[1], (0, EP - E),
                    constant_values=n).astype(jnp.int32)
    w_cat = jnp.concatenate([W_gcn, W_gat], axis=1)
    eye = jnp.eye(H, dtype=x.dtype)
    blk_s = att_src[0][:, :, None] * eye[:, None, :]
    blk_d = att_dst[0][:, :, None] * eye[:, None, :]
    att_mat = jnp.concatenate([blk_s.reshape(H * C, H),
                               blk_d.reshape(H * C, H)], axis=1)

    v_mat = _attvec(W_gat, att_mat)
    feat, aux = _proj(x_p, w_cat, v_mat)
    asrc_p = aux[:, :4]
    adst_p = aux[:, 4:8]
    asrc_f = asrc_p.T.reshape(-1)
    adst_f = adst_p.T.reshape(-1)

    z1 = jnp.zeros((NP // 16, 8), jnp.float32)
    part = _sc_deg_asum(NP, EP, asrc_f, adst_f, row_p, col_p, z1)
    agg = part[:NP] + part[NP:]
    deg = agg[:, 0] + 1.0
    dinv = deg ** -0.5
    zsc = asrc_p + adst_p
    al_self = jnp.maximum(zsc, 0.0) + 0.2 * jnp.minimum(zsc, 0.0)
    p_self = jnp.exp(al_self - jnp.maximum(adst_p, 0.0))
    asum = agg[:, 1:5] + p_self
    rinv = 1.0 / (asum + 1e-16)
    dinv = jnp.where(jnp.arange(NP) < n, dinv, 0.0)
    rinv = jnp.where((jnp.arange(NP) < n)[:, None], rinv, 0.0)

    wflat = _sc_edge_weights(NP, EP, asrc_f, adst_f, rinv.T.reshape(-1),
                             dinv, row_p, col_p)
    wcols = wflat.reshape(5, EP)
    wgt8 = jnp.concatenate(
        [wcols.T, jnp.zeros((EP, 3), jnp.float32)], axis=1)

    z2 = jnp.zeros((24, 1280), jnp.float32)
    out_edge = _sc_aggregate(NP, EP, wgt8, feat, row_p, col_p, z2)

    w_self = p_self * rinv
    out = _fuse(out_edge, feat, (dinv * dinv)[:, None], w_self,
                b_gcn.reshape(1, -1), b_gat.reshape(1, -1),
                W_fuse, b_fuse.reshape(1, -1))
    return out[:n]


# speedup vs baseline: 1.1048x; 1.1048x over previous
import functools

import jax
import jax.numpy as jnp
from jax import lax
from jax.experimental import pallas as pl
from jax.experimental.pallas import tpu as pltpu
from jax.experimental.pallas import tpu_sc as plsc

_L = 16
_NSUB = 32
_BLK_M = 1024


def _proj_body(x_ref, w_ref, v_ref, f_ref, a_ref):
    xb = x_ref[...]
    f_ref[...] = jnp.dot(xb, w_ref[...], preferred_element_type=jnp.float32,
                         precision=lax.Precision.HIGHEST)
    a_ref[...] = jnp.dot(xb, v_ref[...], preferred_element_type=jnp.float32,
                         precision=lax.Precision.HIGHEST)


def _proj(x_p, w_cat, v_mat):
    m, k = x_p.shape
    n = w_cat.shape[1]
    na = v_mat.shape[1]
    grid = (m // _BLK_M,)
    return pl.pallas_call(
        _proj_body,
        grid=grid,
        in_specs=[
            pl.BlockSpec((_BLK_M, k), lambda i: (i, 0)),
            pl.BlockSpec((k, n), lambda i: (0, 0)),
            pl.BlockSpec((k, na), lambda i: (0, 0)),
        ],
        out_specs=[
            pl.BlockSpec((_BLK_M, n), lambda i: (i, 0)),
            pl.BlockSpec((_BLK_M, na), lambda i: (i, 0)),
        ],
        out_shape=[
            jax.ShapeDtypeStruct((m, n), jnp.float32),
            jax.ShapeDtypeStruct((m, na), jnp.float32),
        ],
    )(x_p, w_cat, v_mat)


def _attvec_body(wg_ref, am_ref, o_ref):
    o_ref[...] = jnp.dot(wg_ref[...], am_ref[...],
                         preferred_element_type=jnp.float32,
                         precision=lax.Precision.HIGHEST)


def _attvec(w_gat, att_mat):
    k, n = w_gat.shape
    na = att_mat.shape[1]
    return pl.pallas_call(
        _attvec_body,
        in_specs=[
            pl.BlockSpec((k, n), lambda: (0, 0)),
            pl.BlockSpec((n, na), lambda: (0, 0)),
        ],
        out_specs=pl.BlockSpec((k, na), lambda: (0, 0)),
        out_shape=jax.ShapeDtypeStruct((k, na), jnp.float32),
    )(w_gat, att_mat)


def _fuse_body(oe_ref, ft_ref, d2_ref, ws_ref, bg_ref, bt_ref, wf_ref,
               bf_ref, o_ref):
    agg = oe_ref[...]
    ft = ft_ref[...]
    d2 = d2_ref[...]
    ws = ws_ref[...]
    C = 256
    gcn = agg[:, :C] + d2 * ft[:, :C] + bg_ref[...]
    parts = [gcn]
    for h in range(4):
        lo = C + h * C
        gat_h = (agg[:, lo:lo + C] + ws[:, h:h + 1] * ft[:, lo:lo + C]
                 + bt_ref[:, h * C:(h + 1) * C])
        parts.append(gat_h)
    xcat = jnp.concatenate(parts, axis=1)
    acc = jnp.dot(xcat, wf_ref[...], preferred_element_type=jnp.float32,
                         precision=lax.Precision.HIGHEST)
    acc = acc + bf_ref[...]
    o_ref[...] = jnp.where(acc > 0, acc, jnp.exp(acc) - 1.0)


def _fuse(out_edge, feat, dinv2, w_self, b_gcn, b_gat, w_fuse, b_fuse):
    m, k = out_edge.shape
    n = w_fuse.shape[1]
    grid = (m // _BLK_M,)
    return pl.pallas_call(
        _fuse_body,
        grid=grid,
        in_specs=[
            pl.BlockSpec((_BLK_M, k), lambda i: (i, 0)),
            pl.BlockSpec((_BLK_M, k), lambda i: (i, 0)),
            pl.BlockSpec((_BLK_M, 1), lambda i: (i, 0)),
            pl.BlockSpec((_BLK_M, 4), lambda i: (i, 0)),
            pl.BlockSpec((1, 256), lambda i: (0, 0)),
            pl.BlockSpec((1, 1024), lambda i: (0, 0)),
            pl.BlockSpec((k, n), lambda i: (0, 0)),
            pl.BlockSpec((1, n), lambda i: (0, 0)),
        ],
        out_specs=pl.BlockSpec((_BLK_M, n), lambda i: (i, 0)),
        out_shape=jax.ShapeDtypeStruct((m, n), jnp.float32),
    )(out_edge, feat, dinv2, w_self, b_gcn, b_gat, w_fuse, b_fuse)


_SC_PARAMS = pltpu.CompilerParams(use_tc_tiling_on_sc=False,
                                  needs_layout_passes=False)


def _sc_deg_asum(np_, e_pad, asrc_f, adst_f, row_p, col_p, zrows):
    CH = 128
    slice_len = e_pad // _NSUB
    nchunks = slice_len // CH
    stripe = np_ // 16
    mesh = plsc.VectorSubcoreMesh(core_axis_name="c", subcore_axis_name="s")

    @functools.partial(
        pl.kernel, mesh=mesh,
        compiler_params=_SC_PARAMS,
        out_type=jax.ShapeDtypeStruct((2 * np_, 8), jnp.float32),
        scratch_types=[
            pltpu.VMEM((4 * np_,), jnp.float32),
            pltpu.VMEM((4 * np_,), jnp.float32),
            pltpu.VMEM((CH,), jnp.int32),
            pltpu.VMEM((CH,), jnp.int32),
            pltpu.VMEM((CH, 8), jnp.float32),
            pltpu.VMEM_SHARED((np_, 8), jnp.float32),
        ],
    )
    def k(asrc_h, adst_h, row_h, col_h, z_h, out_h,
          asrc_v, adst_v, rst, cst, sbuf, acc):
        c = lax.axis_index("c")
        s = lax.axis_index("s")
        pltpu.sync_copy(z_h, acc.at[pl.ds(s * stripe, stripe)])
        pltpu.sync_copy(asrc_h, asrc_v)
        pltpu.sync_copy(adst_h, adst_v)
        plsc.subcore_barrier()

        wid = s * 2 + c
        base0 = wid * slice_len
        lane = lax.iota(jnp.int32, 16)
        ones = jnp.ones((16,), jnp.float32)

        def chunk(i, carry):
            b = base0 + i * CH
            pltpu.sync_copy(row_h.at[pl.ds(b, CH)], rst)
            pltpu.sync_copy(col_h.at[pl.ds(b, CH)], cst)
            for g in range(CH // 16):
                rv = rst[pl.ds(g * 16, 16)]
                cv = cst[pl.ds(g * 16, 16)]
                plsc.store_scatter(sbuf, [g * 16 + lane, 0 * lane], ones)
                for h in range(4):
                    sv = plsc.load_gather(asrc_v, [rv + h * np_])
                    dv = plsc.load_gather(adst_v, [cv + h * np_])
                    z = sv + dv
                    al = jnp.maximum(z, 0.0) + 0.2 * jnp.minimum(z, 0.0)
                    p = jnp.exp(al - jnp.maximum(dv, 0.0))
                    plsc.store_scatter(
                        sbuf, [g * 16 + lane, 0 * lane + 1 + h], p)
            pltpu.sync_copy(sbuf, acc.at[cst], add=True)
            return carry

        lax.fori_loop(0, nchunks, chunk, 0)
        plsc.subcore_barrier()
        pltpu.sync_copy(acc.at[pl.ds(s * stripe, stripe)],
                        out_h.at[pl.ds(c * np_ + s * stripe, stripe)])

    return k(asrc_f, adst_f, row_p, col_p, zrows)


def _sc_edge_weights(np_, e_pad, asrc_f, adst_f, rinv_f, dinv_p,
                     row_p, col_p):
    CH = 1024
    slice_len = e_pad // _NSUB
    nchunks = slice_len // CH
    mesh = plsc.VectorSubcoreMesh(core_axis_name="c", subcore_axis_name="s")

    @functools.partial(
        pl.kernel, mesh=mesh,
        compiler_params=_SC_PARAMS,
        out_type=jax.ShapeDtypeStruct((5 * e_pad,), jnp.float32),
        scratch_types=[
            pltpu.VMEM((np_,), jnp.float32),
            pltpu.VMEM((np_,), jnp.float32),
            pltpu.VMEM((np_,), jnp.float32),
            pltpu.VMEM((CH,), jnp.int32),
            pltpu.VMEM((CH,), jnp.int32),
            pltpu.VMEM((CH,), jnp.float32),
        ],
    )
    def k(asrc_h, adst_h, rinv_h, dinv_h, row_h, col_h, out_h,
          ta, tb, tc, rst, cst, obuf):
        c = lax.axis_index("c")
        s = lax.axis_index("s")
        wid = s * 2 + c
        base0 = wid * slice_len
        lane = lax.iota(jnp.int32, 16)

        for ph in range(5):
            if ph == 0:
                pltpu.sync_copy(dinv_h, ta)
            else:
                h = ph - 1
                pltpu.sync_copy(asrc_h.at[pl.ds(h * np_, np_)], ta)
                pltpu.sync_copy(adst_h.at[pl.ds(h * np_, np_)], tb)
                pltpu.sync_copy(rinv_h.at[pl.ds(h * np_, np_)], tc)

            def chunk(i, carry, ph=ph):
                b = base0 + i * CH
                pltpu.sync_copy(row_h.at[pl.ds(b, CH)], rst)
                pltpu.sync_copy(col_h.at[pl.ds(b, CH)], cst)

                def grp(g, carry1):
                    rv = rst[pl.ds(g * 16, 16)]
                    cv = cst[pl.ds(g * 16, 16)]
                    if ph == 0:
                        wv = (plsc.load_gather(ta, [rv])
                              * plsc.load_gather(ta, [cv]))
                    else:
                        sv = plsc.load_gather(ta, [rv])
                        dv = plsc.load_gather(tb, [cv])
                        ri = plsc.load_gather(tc, [cv])
                        z = sv + dv
                        al = jnp.maximum(z, 0.0) + 0.2 * jnp.minimum(z, 0.0)
                        wv = jnp.exp(al - jnp.maximum(dv, 0.0)) * ri
                    plsc.store_scatter(obuf, [g * 16 + lane], wv)
                    return carry1

                lax.fori_loop(0, CH // 16, grp, 0)
                pltpu.sync_copy(obuf, out_h.at[pl.ds(ph * e_pad + b, CH)])
                return carry

            lax.fori_loop(0, nchunks, chunk, 0)

    return k(asrc_f, adst_f, rinv_f, dinv_p, row_p, col_p)


def _sc_aggregate(np_, e_pad, wgt8, feat, row_p, col_p, zrows):
    W = 512
    D = 1280
    NPASS = np_ // (W * 2)
    CH = 2048
    slice_len = e_pad // 16
    nchunks = slice_len // CH
    cap = 2112
    stripe = W // 16
    mesh = plsc.VectorSubcoreMesh(core_axis_name="c", subcore_axis_name="s")

    @functools.partial(
        pl.kernel, mesh=mesh,
        compiler_params=_SC_PARAMS,
        out_type=jax.ShapeDtypeStruct((np_, D), jnp.float32),
        scratch_types=[
            pltpu.VMEM((CH,), jnp.int32),
            pltpu.VMEM((CH,), jnp.int32),
            pltpu.VMEM((cap,), jnp.int32),
            pltpu.VMEM((cap,), jnp.int32),
            pltpu.VMEM((cap,), jnp.int32),
            pltpu.VMEM((16, 8), jnp.float32),
            pltpu.VMEM((16, 8), jnp.float32),
            pltpu.VMEM((16, D), jnp.float32),
            pltpu.VMEM((16, D), jnp.float32),
            pltpu.VMEM((16, D), jnp.float32),
            pltpu.VMEM_SHARED((W, D), jnp.float32),
            pltpu.SemaphoreType.DMA,
            pltpu.SemaphoreType.DMA,
            pltpu.SemaphoreType.DMA,
            pltpu.SemaphoreType.DMA,
            pltpu.SemaphoreType.DMA,
        ],
    )
    def k(wgt_h, feat_h, row_h, col_h, z_h, out_h,
          rst, cst, rbuf, cbuf, ebuf, wbufA, wbufB, fbufA, fbufB, sbuf,
          acc, fsemA, fsemB, wsemA, wsemB, ssem):
        c = lax.axis_index("c")
        s = lax.axis_index("s")
        lane = lax.iota(jnp.int32, 16)
        base0 = s * slice_len

        def zinit(i, carry):
            rbuf[pl.ds(i * 16, 16)] = 0 * lane
            cbuf[pl.ds(i * 16, 16)] = 0 * lane
            ebuf[pl.ds(i * 16, 16)] = 0 * lane
            return carry

        lax.fori_loop(0, cap // 16, zinit, 0)

        def scale(wb, fb, j, cnt):
            valid = ((j * 16 + lane) < cnt).astype(jnp.float32)
            for kk in range(5):
                wk = plsc.load_gather(wb, [lane, 0 * lane + kk])
                plsc.store_scatter(wb, [lane, 0 * lane + kk], wk * valid)

            def edge(jj, carry3):
                jv = 0 * lane + jj
                gs = plsc.load_gather(wb, [jv, 0 * lane])
                hs = [plsc.load_gather(wb, [jv, 0 * lane + 1 + h])
                      for h in range(4)]
                for v in range(D // 16):
                    wv = gs if v < 16 else hs[(v - 16) // 16]
                    xv = fb[jj, pl.ds(v * 16, 16)]
                    sbuf[jj, pl.ds(v * 16, 16)] = xv * wv
                return carry3

            lax.fori_loop(0, 16, edge, 0)

        def issue(jg, wb, fb, wsem, fsem):
            ev = ebuf[pl.ds(jg * 16, 16)]
            rv = rbuf[pl.ds(jg * 16, 16)]
            pltpu.async_copy(wgt_h.at[ev], wb, wsem)
            pltpu.async_copy(feat_h.at[rv], fb, fsem)

        def wait_g(wb, fb, wsem, fsem):
            ev0 = ebuf[pl.ds(0, 16)]
            rv0 = rbuf[pl.ds(0, 16)]
            pltpu.make_async_copy(wgt_h.at[ev0], wb, wsem).wait()
            pltpu.make_async_copy(feat_h.at[rv0], fb, fsem).wait()

        def one_pass(p, carry):
            wlo = (p * 2 + c) * W
            pltpu.sync_copy(z_h, acc.at[pl.ds(s * stripe, stripe)])
            plsc.subcore_barrier()

            def chunk(i, carry1):
                b = base0 + i * CH
                pltpu.sync_copy(row_h.at[pl.ds(b, CH)], rst)
                pltpu.sync_copy(col_h.at[pl.ds(b, CH)], cst)

                def scan_grp(g, cnt):
                    rv = rst[pl.ds(g * 16, 16)]
                    cv = cst[pl.ds(g * 16, 16)]
                    rel = cv - wlo
                    m = (rel >= 0) & (rel < W)
                    plsc.store_compressed(rbuf.at[pl.ds(cnt, 16)], rv,
                                          mask=m)
                    plsc.store_compressed(cbuf.at[pl.ds(cnt, 16)], rel,
                                          mask=m)
                    plsc.store_compressed(ebuf.at[pl.ds(cnt, 16)],
                                          b + g * 16 + lane, mask=m)
                    return cnt + jnp.max(
                        plsc.all_reduce_population_count(m))

                cnt = lax.fori_loop(0, CH // 16, scan_grp, jnp.int32(0))
                rbuf[pl.ds(cnt, 16)] = 0 * lane
                cbuf[pl.ds(cnt, 16)] = 0 * lane
                ebuf[pl.ds(cnt, 16)] = 0 * lane

                issue(0, wbufA, fbufA, wsemA, fsemA)

                def grp2(jp, carry2):
                    jA = jp * 2
                    jB = jp * 2 + 1
                    issue(jB, wbufB, fbufB, wsemB, fsemB)
                    wait_g(wbufA, fbufA, wsemA, fsemA)
                    scale(wbufA, fbufA, jA, cnt)
                    relcA = cbuf[pl.ds(jA * 16, 16)]
                    dA = pltpu.async_copy(sbuf, acc.at[relcA], ssem,
                                          add=True)
                    issue(jA + 2, wbufA, fbufA, wsemA, fsemA)
                    wait_g(wbufB, fbufB, wsemB, fsemB)
                    dA.wait()
                    scale(wbufB, fbufB, jB, cnt)
                    relcB = cbuf[pl.ds(jB * 16, 16)]
                    pltpu.async_copy(sbuf, acc.at[relcB], ssem,
                                     add=True).wait()
                    return carry2

                lax.fori_loop(0, (cnt + 31) // 32, grp2, 0)
                wait_g(wbufA, fbufA, wsemA, fsemA)
                return carry1

            lax.fori_loop(0, nchunks, chunk, 0)
            plsc.subcore_barrier()
            pltpu.sync_copy(acc.at[pl.ds(s * stripe, stripe)],
                            out_h.at[pl.ds(wlo + s * stripe, stripe)])
            return carry

        lax.fori_loop(0, NPASS, one_pass, 0)

    return k(wgt8, feat, row_p, col_p, zrows)


def kernel(x, edge_index, W_gcn, b_gcn, W_gat, att_src, att_dst, b_gat,
           W_fuse, b_fuse):
    n, d_in = x.shape
    H = att_src.shape[1]
    C = att_src.shape[2]
    E = edge_index.shape[1]
    NP = 12288
    EP = ((E + 4095) // 4096) * 4096

    x_p = jnp.pad(x, ((0, NP - n), (0, 0)))
    row_p = jnp.pad(edge_index[0], (0, EP - E)).astype(jnp.int32)
    col_p = jnp.pad(edge_index[1], (0, EP - E),
                    constant_values=n).astype(jnp.int32)
    w_cat = jnp.concatenate([W_gcn, W_gat], axis=1)
    eye = jnp.eye(H, dtype=x.dtype)
    blk_s = att_src[0][:, :, None] * eye[:, None, :]
    blk_d = att_dst[0][:, :, None] * eye[:, None, :]
    att_mat = jnp.concatenate([blk_s.reshape(H * C, H),
                               blk_d.reshape(H * C, H)], axis=1)

    v_mat = _attvec(W_gat, att_mat)
    feat, aux = _proj(x_p, w_cat, v_mat)
    asrc_p = aux[:, :4]
    adst_p = aux[:, 4:8]
    asrc_f = asrc_p.T.reshape(-1)
    adst_f = adst_p.T.reshape(-1)

    z1 = jnp.zeros((NP // 16, 8), jnp.float32)
    part = _sc_deg_asum(NP, EP, asrc_f, adst_f, row_p, col_p, z1)
    agg = part[:NP] + part[NP:]
    deg = agg[:, 0] + 1.0
    dinv = deg ** -0.5
    zsc = asrc_p + adst_p
    al_self = jnp.maximum(zsc, 0.0) + 0.2 * jnp.minimum(zsc, 0.0)
    p_self = jnp.exp(al_self - jnp.maximum(adst_p, 0.0))
    asum = agg[:, 1:5] + p_self
    rinv = 1.0 / (asum + 1e-16)
    dinv = jnp.where(jnp.arange(NP) < n, dinv, 0.0)
    rinv = jnp.where((jnp.arange(NP) < n)[:, None], rinv, 0.0)

    wflat = _sc_edge_weights(NP, EP, asrc_f, adst_f, rinv.T.reshape(-1),
                             dinv, row_p, col_p)
    wcols = wflat.reshape(5, EP)
    wgt8 = jnp.concatenate(
        [wcols.T, jnp.zeros((EP, 3), jnp.float32)], axis=1)

    z2 = jnp.zeros((32, 1280), jnp.float32)
    out_edge = _sc_aggregate(NP, EP, wgt8, feat, row_p, col_p, z2)

    w_self = p_self * rinv
    out = _fuse(out_edge, feat, (dinv * dinv)[:, None], w_self,
                b_gcn.reshape(1, -1), b_gat.reshape(1, -1),
                W_fuse, b_fuse.reshape(1, -1))
    return out[:n]


# speedup vs baseline: 1.1563x; 1.0466x over previous
import functools

import jax
import jax.numpy as jnp
from jax import lax
from jax.experimental import pallas as pl
from jax.experimental.pallas import tpu as pltpu
from jax.experimental.pallas import tpu_sc as plsc

_L = 16
_NSUB = 32
_BLK_M = 1024


def _proj_body(x_ref, w_ref, v_ref, f_ref, a_ref):
    xb = x_ref[...]
    f_ref[...] = jnp.dot(xb, w_ref[...], preferred_element_type=jnp.float32,
                         precision=lax.Precision.HIGHEST)
    a_ref[...] = jnp.dot(xb, v_ref[...], preferred_element_type=jnp.float32,
                         precision=lax.Precision.HIGHEST)


def _proj(x_p, w_cat, v_mat):
    m, k = x_p.shape
    n = w_cat.shape[1]
    na = v_mat.shape[1]
    grid = (m // _BLK_M,)
    return pl.pallas_call(
        _proj_body,
        grid=grid,
        in_specs=[
            pl.BlockSpec((_BLK_M, k), lambda i: (i, 0)),
            pl.BlockSpec((k, n), lambda i: (0, 0)),
            pl.BlockSpec((k, na), lambda i: (0, 0)),
        ],
        out_specs=[
            pl.BlockSpec((_BLK_M, n), lambda i: (i, 0)),
            pl.BlockSpec((_BLK_M, na), lambda i: (i, 0)),
        ],
        out_shape=[
            jax.ShapeDtypeStruct((m, n), jnp.float32),
            jax.ShapeDtypeStruct((m, na), jnp.float32),
        ],
    )(x_p, w_cat, v_mat)


def _attvec_body(wg_ref, am_ref, o_ref):
    o_ref[...] = jnp.dot(wg_ref[...], am_ref[...],
                         preferred_element_type=jnp.float32,
                         precision=lax.Precision.HIGHEST)


def _attvec(w_gat, att_mat):
    k, n = w_gat.shape
    na = att_mat.shape[1]
    return pl.pallas_call(
        _attvec_body,
        in_specs=[
            pl.BlockSpec((k, n), lambda: (0, 0)),
            pl.BlockSpec((n, na), lambda: (0, 0)),
        ],
        out_specs=pl.BlockSpec((k, na), lambda: (0, 0)),
        out_shape=jax.ShapeDtypeStruct((k, na), jnp.float32),
    )(w_gat, att_mat)


def _fuse_body(oe_ref, ft_ref, d2_ref, ws_ref, bg_ref, bt_ref, wf_ref,
               bf_ref, o_ref):
    agg = oe_ref[...]
    ft = ft_ref[...]
    d2 = d2_ref[...]
    ws = ws_ref[...]
    C = 256
    gcn = agg[:, :C] + d2 * ft[:, :C] + bg_ref[...]
    parts = [gcn]
    for h in range(4):
        lo = C + h * C
        gat_h = (agg[:, lo:lo + C] + ws[:, h:h + 1] * ft[:, lo:lo + C]
                 + bt_ref[:, h * C:(h + 1) * C])
        parts.append(gat_h)
    xcat = jnp.concatenate(parts, axis=1)
    acc = jnp.dot(xcat, wf_ref[...], preferred_element_type=jnp.float32,
                         precision=lax.Precision.HIGHEST)
    acc = acc + bf_ref[...]
    o_ref[...] = jnp.where(acc > 0, acc, jnp.exp(acc) - 1.0)


def _fuse(out_edge, feat, dinv2, w_self, b_gcn, b_gat, w_fuse, b_fuse):
    m, k = out_edge.shape
    n = w_fuse.shape[1]
    grid = (m // _BLK_M,)
    return pl.pallas_call(
        _fuse_body,
        grid=grid,
        in_specs=[
            pl.BlockSpec((_BLK_M, k), lambda i: (i, 0)),
            pl.BlockSpec((_BLK_M, k), lambda i: (i, 0)),
            pl.BlockSpec((_BLK_M, 1), lambda i: (i, 0)),
            pl.BlockSpec((_BLK_M, 4), lambda i: (i, 0)),
            pl.BlockSpec((1, 256), lambda i: (0, 0)),
            pl.BlockSpec((1, 1024), lambda i: (0, 0)),
            pl.BlockSpec((k, n), lambda i: (0, 0)),
            pl.BlockSpec((1, n), lambda i: (0, 0)),
        ],
        out_specs=pl.BlockSpec((_BLK_M, n), lambda i: (i, 0)),
        out_shape=jax.ShapeDtypeStruct((m, n), jnp.float32),
    )(out_edge, feat, dinv2, w_self, b_gcn, b_gat, w_fuse, b_fuse)


_SC_PARAMS = pltpu.CompilerParams(use_tc_tiling_on_sc=False,
                                  needs_layout_passes=False)


def _sc_deg_asum(np_, e_pad, asrc_f, adst_f, row_p, col_p, zrows):
    CH = 128
    slice_len = e_pad // _NSUB
    nchunks = slice_len // CH
    stripe = np_ // 16
    mesh = plsc.VectorSubcoreMesh(core_axis_name="c", subcore_axis_name="s")

    @functools.partial(
        pl.kernel, mesh=mesh,
        compiler_params=_SC_PARAMS,
        out_type=jax.ShapeDtypeStruct((2 * np_, 8), jnp.float32),
        scratch_types=[
            pltpu.VMEM((4 * np_,), jnp.float32),
            pltpu.VMEM((4 * np_,), jnp.float32),
            pltpu.VMEM((CH,), jnp.int32),
            pltpu.VMEM((CH,), jnp.int32),
            pltpu.VMEM((CH, 8), jnp.float32),
            pltpu.VMEM_SHARED((np_, 8), jnp.float32),
        ],
    )
    def k(asrc_h, adst_h, row_h, col_h, z_h, out_h,
          asrc_v, adst_v, rst, cst, sbuf, acc):
        c = lax.axis_index("c")
        s = lax.axis_index("s")
        pltpu.sync_copy(z_h, acc.at[pl.ds(s * stripe, stripe)])
        pltpu.sync_copy(asrc_h, asrc_v)
        pltpu.sync_copy(adst_h, adst_v)
        plsc.subcore_barrier()

        wid = s * 2 + c
        base0 = wid * slice_len
        lane = lax.iota(jnp.int32, 16)
        ones = jnp.ones((16,), jnp.float32)

        def chunk(i, carry):
            b = base0 + i * CH
            pltpu.sync_copy(row_h.at[pl.ds(b, CH)], rst)
            pltpu.sync_copy(col_h.at[pl.ds(b, CH)], cst)
            for g in range(CH // 16):
                rv = rst[pl.ds(g * 16, 16)]
                cv = cst[pl.ds(g * 16, 16)]
                plsc.store_scatter(sbuf, [g * 16 + lane, 0 * lane], ones)
                for h in range(4):
                    sv = plsc.load_gather(asrc_v, [rv + h * np_])
                    dv = plsc.load_gather(adst_v, [cv + h * np_])
                    z = sv + dv
                    al = jnp.maximum(z, 0.0) + 0.2 * jnp.minimum(z, 0.0)
                    p = jnp.exp(al - jnp.maximum(dv, 0.0))
                    plsc.store_scatter(
                        sbuf, [g * 16 + lane, 0 * lane + 1 + h], p)
            pltpu.sync_copy(sbuf, acc.at[cst], add=True)
            return carry

        lax.fori_loop(0, nchunks, chunk, 0)
        plsc.subcore_barrier()
        pltpu.sync_copy(acc.at[pl.ds(s * stripe, stripe)],
                        out_h.at[pl.ds(c * np_ + s * stripe, stripe)])

    return k(asrc_f, adst_f, row_p, col_p, zrows)


def _sc_edge_weights(np_, e_pad, asrc_f, adst_f, rinv_f, dinv_p,
                     row_p, col_p):
    CH = 1024
    slice_len = e_pad // _NSUB
    nchunks = slice_len // CH
    mesh = plsc.VectorSubcoreMesh(core_axis_name="c", subcore_axis_name="s")

    @functools.partial(
        pl.kernel, mesh=mesh,
        compiler_params=_SC_PARAMS,
        out_type=jax.ShapeDtypeStruct((5 * e_pad,), jnp.float32),
        scratch_types=[
            pltpu.VMEM((np_,), jnp.float32),
            pltpu.VMEM((np_,), jnp.float32),
            pltpu.VMEM((np_,), jnp.float32),
            pltpu.VMEM((CH,), jnp.int32),
            pltpu.VMEM((CH,), jnp.int32),
            pltpu.VMEM((CH,), jnp.float32),
        ],
    )
    def k(asrc_h, adst_h, rinv_h, dinv_h, row_h, col_h, out_h,
          ta, tb, tc, rst, cst, obuf):
        c = lax.axis_index("c")
        s = lax.axis_index("s")
        wid = s * 2 + c
        base0 = wid * slice_len
        lane = lax.iota(jnp.int32, 16)

        for ph in range(5):
            if ph == 0:
                pltpu.sync_copy(dinv_h, ta)
            else:
                h = ph - 1
                pltpu.sync_copy(asrc_h.at[pl.ds(h * np_, np_)], ta)
                pltpu.sync_copy(adst_h.at[pl.ds(h * np_, np_)], tb)
                pltpu.sync_copy(rinv_h.at[pl.ds(h * np_, np_)], tc)

            def chunk(i, carry, ph=ph):
                b = base0 + i * CH
                pltpu.sync_copy(row_h.at[pl.ds(b, CH)], rst)
                pltpu.sync_copy(col_h.at[pl.ds(b, CH)], cst)

                def grp(g, carry1):
                    rv = rst[pl.ds(g * 16, 16)]
                    cv = cst[pl.ds(g * 16, 16)]
                    if ph == 0:
                        wv = (plsc.load_gather(ta, [rv])
                              * plsc.load_gather(ta, [cv]))
                    else:
                        sv = plsc.load_gather(ta, [rv])
                        dv = plsc.load_gather(tb, [cv])
                        ri = plsc.load_gather(tc, [cv])
                        z = sv + dv
                        al = jnp.maximum(z, 0.0) + 0.2 * jnp.minimum(z, 0.0)
                        wv = jnp.exp(al - jnp.maximum(dv, 0.0)) * ri
                    plsc.store_scatter(obuf, [g * 16 + lane], wv)
                    return carry1

                lax.fori_loop(0, CH // 16, grp, 0)
                pltpu.sync_copy(obuf, out_h.at[pl.ds(ph * e_pad + b, CH)])
                return carry

            lax.fori_loop(0, nchunks, chunk, 0)

    return k(asrc_f, adst_f, rinv_f, dinv_p, row_p, col_p)


def _sc_aggregate(np_, e_pad, wgt8, feat, row_p, col_p, zrows):
    W = 768
    D = 1280
    NPASS = np_ // (W * 2)
    CH = 1280
    slice_len = e_pad // 16
    nchunks = slice_len // CH
    cap = 1344
    stripe = W // 16
    mesh = plsc.VectorSubcoreMesh(core_axis_name="c", subcore_axis_name="s")

    @functools.partial(
        pl.kernel, mesh=mesh,
        compiler_params=_SC_PARAMS,
        out_type=jax.ShapeDtypeStruct((np_, D), jnp.float32),
        scratch_types=[
            pltpu.VMEM((CH,), jnp.int32),
            pltpu.VMEM((CH,), jnp.int32),
            pltpu.VMEM((cap,), jnp.int32),
            pltpu.VMEM((cap,), jnp.int32),
            pltpu.VMEM((cap,), jnp.int32),
            pltpu.VMEM((16, 8), jnp.float32),
            pltpu.VMEM((16, 8), jnp.float32),
            pltpu.VMEM((16, D), jnp.float32),
            pltpu.VMEM((16, D), jnp.float32),
            pltpu.VMEM((16, D), jnp.float32),
            pltpu.VMEM_SHARED((W, D), jnp.float32),
            pltpu.SemaphoreType.DMA,
            pltpu.SemaphoreType.DMA,
            pltpu.SemaphoreType.DMA,
            pltpu.SemaphoreType.DMA,
            pltpu.SemaphoreType.DMA,
        ],
    )
    def k(wgt_h, feat_h, row_h, col_h, z_h, out_h,
          rst, cst, rbuf, cbuf, ebuf, wbufA, wbufB, fbufA, fbufB, sbuf,
          acc, fsemA, fsemB, wsemA, wsemB, ssem):
        c = lax.axis_index("c")
        s = lax.axis_index("s")
        lane = lax.iota(jnp.int32, 16)
        base0 = s * slice_len

        def zinit(i, carry):
            rbuf[pl.ds(i * 16, 16)] = 0 * lane
            cbuf[pl.ds(i * 16, 16)] = 0 * lane
            ebuf[pl.ds(i * 16, 16)] = 0 * lane
            return carry

        lax.fori_loop(0, cap // 16, zinit, 0)

        def scale(wb, fb, j, cnt):
            valid = ((j * 16 + lane) < cnt).astype(jnp.float32)
            for kk in range(5):
                wk = plsc.load_gather(wb, [lane, 0 * lane + kk])
                plsc.store_scatter(wb, [lane, 0 * lane + kk], wk * valid)

            def edge(jj, carry3):
                jv = 0 * lane + jj
                gs = plsc.load_gather(wb, [jv, 0 * lane])
                hs = [plsc.load_gather(wb, [jv, 0 * lane + 1 + h])
                      for h in range(4)]
                for v in range(D // 16):
                    wv = gs if v < 16 else hs[(v - 16) // 16]
                    xv = fb[jj, pl.ds(v * 16, 16)]
                    sbuf[jj, pl.ds(v * 16, 16)] = xv * wv
                return carry3

            lax.fori_loop(0, 16, edge, 0)

        def issue(jg, wb, fb, wsem, fsem):
            ev = ebuf[pl.ds(jg * 16, 16)]
            rv = rbuf[pl.ds(jg * 16, 16)]
            pltpu.async_copy(wgt_h.at[ev], wb, wsem)
            pltpu.async_copy(feat_h.at[rv], fb, fsem)

        def wait_g(wb, fb, wsem, fsem):
            ev0 = ebuf[pl.ds(0, 16)]
            rv0 = rbuf[pl.ds(0, 16)]
            pltpu.make_async_copy(wgt_h.at[ev0], wb, wsem).wait()
            pltpu.make_async_copy(feat_h.at[rv0], fb, fsem).wait()

        def one_pass(p, carry):
            wlo = (p * 2 + c) * W
            pltpu.sync_copy(z_h, acc.at[pl.ds(s * stripe, stripe)])
            plsc.subcore_barrier()

            def chunk(i, carry1):
                b = base0 + i * CH
                pltpu.sync_copy(row_h.at[pl.ds(b, CH)], rst)
                pltpu.sync_copy(col_h.at[pl.ds(b, CH)], cst)

                def scan_grp(g, cnt):
                    rv = rst[pl.ds(g * 16, 16)]
                    cv = cst[pl.ds(g * 16, 16)]
                    rel = cv - wlo
                    m = (rel >= 0) & (rel < W)
                    plsc.store_compressed(rbuf.at[pl.ds(cnt, 16)], rv,
                                          mask=m)
                    plsc.store_compressed(cbuf.at[pl.ds(cnt, 16)], rel,
                                          mask=m)
                    plsc.store_compressed(ebuf.at[pl.ds(cnt, 16)],
                                          b + g * 16 + lane, mask=m)
                    return cnt + jnp.max(
                        plsc.all_reduce_population_count(m))

                cnt = lax.fori_loop(0, CH // 16, scan_grp, jnp.int32(0))
                rbuf[pl.ds(cnt, 16)] = 0 * lane
                cbuf[pl.ds(cnt, 16)] = 0 * lane
                ebuf[pl.ds(cnt, 16)] = 0 * lane

                issue(0, wbufA, fbufA, wsemA, fsemA)

                def grp2(jp, carry2):
                    jA = jp * 2
                    jB = jp * 2 + 1
                    issue(jB, wbufB, fbufB, wsemB, fsemB)
                    wait_g(wbufA, fbufA, wsemA, fsemA)
                    scale(wbufA, fbufA, jA, cnt)
                    relcA = cbuf[pl.ds(jA * 16, 16)]
                    dA = pltpu.async_copy(sbuf, acc.at[relcA], ssem,
                                          add=True)
                    issue(jA + 2, wbufA, fbufA, wsemA, fsemA)
                    wait_g(wbufB, fbufB, wsemB, fsemB)
                    dA.wait()
                    scale(wbufB, fbufB, jB, cnt)
                    relcB = cbuf[pl.ds(jB * 16, 16)]
                    pltpu.async_copy(sbuf, acc.at[relcB], ssem,
                                     add=True).wait()
                    return carry2

                lax.fori_loop(0, (cnt + 31) // 32, grp2, 0)
                wait_g(wbufA, fbufA, wsemA, fsemA)
                return carry1

            lax.fori_loop(0, nchunks, chunk, 0)
            plsc.subcore_barrier()
            pltpu.sync_copy(acc.at[pl.ds(s * stripe, stripe)],
                            out_h.at[pl.ds(wlo + s * stripe, stripe)])
            return carry

        lax.fori_loop(0, NPASS, one_pass, 0)

    return k(wgt8, feat, row_p, col_p, zrows)


def kernel(x, edge_index, W_gcn, b_gcn, W_gat, att_src, att_dst, b_gat,
           W_fuse, b_fuse):
    n, d_in = x.shape
    H = att_src.shape[1]
    C = att_src.shape[2]
    E = edge_index.shape[1]
    NP = 12288
    EP = ((E + 4095) // 4096) * 4096

    x_p = jnp.pad(x, ((0, NP - n), (0, 0)))
    row_p = jnp.pad(edge_index[0], (0, EP - E)).astype(jnp.int32)
    col_p = jnp.pad(edge_index[1], (0, EP - E),
                    constant_values=n).astype(jnp.int32)
    w_cat = jnp.concatenate([W_gcn, W_gat], axis=1)
    eye = jnp.eye(H, dtype=x.dtype)
    blk_s = att_src[0][:, :, None] * eye[:, None, :]
    blk_d = att_dst[0][:, :, None] * eye[:, None, :]
    att_mat = jnp.concatenate([blk_s.reshape(H * C, H),
                               blk_d.reshape(H * C, H)], axis=1)

    v_mat = _attvec(W_gat, att_mat)
    feat, aux = _proj(x_p, w_cat, v_mat)
    asrc_p = aux[:, :4]
    adst_p = aux[:, 4:8]
    asrc_f = asrc_p.T.reshape(-1)
    adst_f = adst_p.T.reshape(-1)

    z1 = jnp.zeros((NP // 16, 8), jnp.float32)
    part = _sc_deg_asum(NP, EP, asrc_f, adst_f, row_p, col_p, z1)
    agg = part[:NP] + part[NP:]
    deg = agg[:, 0] + 1.0
    dinv = deg ** -0.5
    zsc = asrc_p + adst_p
    al_self = jnp.maximum(zsc, 0.0) + 0.2 * jnp.minimum(zsc, 0.0)
    p_self = jnp.exp(al_self - jnp.maximum(adst_p, 0.0))
    asum = agg[:, 1:5] + p_self
    rinv = 1.0 / (asum + 1e-16)
    dinv = jnp.where(jnp.arange(NP) < n, dinv, 0.0)
    rinv = jnp.where((jnp.arange(NP) < n)[:, None], rinv, 0.0)

    wflat = _sc_edge_weights(NP, EP, asrc_f, adst_f, rinv.T.reshape(-1),
                             dinv, row_p, col_p)
    wcols = wflat.reshape(5, EP)
    wgt8 = jnp.concatenate(
        [wcols.T, jnp.zeros((EP, 3), jnp.float32)], axis=1)

    z2 = jnp.zeros((48, 1280), jnp.float32)
    out_edge = _sc_aggregate(NP, EP, wgt8, feat, row_p, col_p, z2)

    w_self = p_self * rinv
    out = _fuse(out_edge, feat, (dinv * dinv)[:, None], w_self,
                b_gcn.reshape(1, -1), b_gat.reshape(1, -1),
                W_fuse, b_fuse.reshape(1, -1))
    return out[:n]
